# Initial kernel scaffold; baseline (speedup 1.0000x reference)
import functools

import jax
import jax.numpy as jnp
from jax import lax
from jax.experimental import pallas as pl
from jax.experimental.pallas import tpu as pltpu

B = 64
N = 128
E = 2048
G = 4


def _graph_body(eit_ref, ei_ref, x_ref,
                W1a_ref, b1a_ref, W1b_ref, b1b_ref,
                W2a_ref, b2a_ref, W2b_ref, b2b_ref,
                W3a_ref, b3a_ref, W3b_ref, b3b_ref,
                g1_ref, be1_ref, g2_ref, be2_ref, g3_ref, be3_ref,
                out_ref):
    f32 = jnp.float32
    src_col = eit_ref[:, 0:1]
    dst_col = eit_ref[:, 1:2]
    dst_row = ei_ref[1:2, :]
    niota_r = lax.broadcasted_iota(jnp.int32, (E, N), 1)
    D = (dst_col == niota_r).astype(f32)
    S = (src_col == niota_r).astype(f32)
    niota_c = lax.broadcasted_iota(jnp.int32, (N, E), 0)
    DT = (niota_c == dst_row).astype(f32)
    deg = jnp.sum(DT, axis=1, keepdims=True)

    def layer(h, fin, Wa_ref, ba_ref, Wb_ref, bb_ref, g_ref, be_ref):
        Wa = Wa_ref[...]
        Wd = Wa[:fin] - Wa[fin:]
        Wbot = Wa[fin:]
        ba = ba_ref[...]
        hf = h.reshape(B * N, fin)
        Av = (jnp.dot(hf, Wd, preferred_element_type=f32) + ba).reshape(B, N, 64)
        Bv = jnp.dot(hf, Wbot, preferred_element_type=f32).reshape(B, N, 64)

        Wb = Wb_ref[...]
        Z = jnp.zeros((64, 128), f32)
        WbG = jnp.concatenate(
            [jnp.concatenate([Wb if j == i else Z for j in range(G)], axis=1)
             for i in range(G)], axis=0)
        bb = bb_ref[...]
        bbG = jnp.concatenate([bb] * G, axis=1)

        def grp(i, out_acc):
            a = jnp.concatenate(
                [lax.dynamic_slice(Av, (i * G + j, 0, 0), (1, N, 64)).reshape(N, 64)
                 for j in range(G)], axis=1)
            bm = jnp.concatenate(
                [lax.dynamic_slice(Bv, (i * G + j, 0, 0), (1, N, 64)).reshape(N, 64)
                 for j in range(G)], axis=1)
            pre = (jnp.dot(D, a, preferred_element_type=f32)
                   + jnp.dot(S, bm, preferred_element_type=f32))
            r = jnp.maximum(pre, 0.0)
            agg = jnp.dot(DT, r, preferred_element_type=f32)
            o = jnp.dot(agg, WbG, preferred_element_type=f32) + deg * bbG
            for j in range(G):
                out_acc = lax.dynamic_update_slice(
                    out_acc, o[:, j * 128:(j + 1) * 128].reshape(1, N, 128),
                    (i * G + j, 0, 0))
            return out_acc

        out = lax.fori_loop(0, B // G, grp, jnp.zeros((B, N, 128), f32))
        inv = 1.0 / (B * 128)
        mu = jnp.sum(jnp.sum(out, axis=0, keepdims=True), axis=2, keepdims=True) * inv
        d = out - mu
        var = jnp.sum(jnp.sum(d * d, axis=0, keepdims=True), axis=2, keepdims=True) * inv
        hn = d * lax.rsqrt(var + 1e-5) * g_ref[...] + be_ref[...]
        return jnp.maximum(hn, 0.0)

    h = layer(x_ref[...], 2, W1a_ref, b1a_ref, W1b_ref, b1b_ref, g1_ref, be1_ref)
    h = layer(h, 128, W2a_ref, b2a_ref, W2b_ref, b2b_ref, g2_ref, be2_ref)
    h = layer(h, 128, W3a_ref, b3a_ref, W3b_ref, b3b_ref, g3_ref, be3_ref)
    out_ref[...] = h


def _head_body(hf_ref, Wm1_ref, bm1_ref, Wm2_ref, bm2_ref, Wm3_ref, bm3_ref,
               Wv_ref, bv_ref, Wadv_ref, badv_ref, out_ref, acc_ref):
    f32 = jnp.float32
    k = pl.program_id(0)

    @pl.when(k == 0)
    def _():
        acc_ref[...] = jnp.zeros_like(acc_ref)

    acc_ref[...] += jnp.dot(hf_ref[...], Wm1_ref[...], preferred_element_type=f32)

    @pl.when(k == pl.num_programs(0) - 1)
    def _():
        z = jnp.maximum(acc_ref[...] + bm1_ref[...], 0.0)
        z = jnp.maximum(jnp.dot(z, Wm2_ref[...], preferred_element_type=f32) + bm2_ref[...], 0.0)
        z = jnp.maximum(jnp.dot(z, Wm3_ref[...], preferred_element_type=f32) + bm3_ref[...], 0.0)
        value = jnp.dot(z, Wv_ref[...], preferred_element_type=f32) + bv_ref[...]
        adv = jnp.dot(z, Wadv_ref[...], preferred_element_type=f32) + badv_ref[...]
        ii = lax.broadcasted_iota(jnp.int32, (64, 64), 0) // 2
        jj = lax.broadcasted_iota(jnp.int32, (64, 64), 1) // 2
        P = 0.5 * (ii == jj).astype(f32)
        out_ref[...] = value + adv - jnp.dot(adv, P, preferred_element_type=f32)


def kernel(x, edge_index, W1a, b1a, W1b, b1b, W2a, b2a, W2b, b2b,
           W3a, b3a, W3b, b3b, g1, be1, g2, be2, g3, be3,
           Wm1, bm1, Wm2, bm2, Wm3, bm3, Wv, bv, Wadv, badv):
    f32 = jnp.float32
    eit = edge_index.T
    h3 = pl.pallas_call(
        _graph_body,
        out_shape=jax.ShapeDtypeStruct((B, N, 128), f32),
    )(eit, edge_index, x,
      W1a, b1a.reshape(1, 64), W1b, b1b.reshape(1, 128),
      W2a, b2a.reshape(1, 64), W2b, b2b.reshape(1, 128),
      W3a, b3a.reshape(1, 64), W3b, b3b.reshape(1, 128),
      g1.reshape(1, N, 1), be1.reshape(1, N, 1),
      g2.reshape(1, N, 1), be2.reshape(1, N, 1),
      g3.reshape(1, N, 1), be3.reshape(1, N, 1))

    hf = h3.reshape(B, N * N)
    Wadv_r = jnp.transpose(Wadv, (1, 0, 2)).reshape(256, 64)
    KC = 8
    KW = (N * N) // KC

    def cst(shape):
        return pl.BlockSpec(shape, lambda k: tuple(0 for _ in shape))

    q = pl.pallas_call(
        _head_body,
        grid=(KC,),
        in_specs=[
            pl.BlockSpec((B, KW), lambda k: (0, k)),
            pl.BlockSpec((KW, 256), lambda k: (k, 0)),
            cst((1, 256)), cst((256, 256)), cst((1, 256)),
            cst((256, 256)), cst((1, 256)),
            cst((256, 1)), cst((1, 1)), cst((256, 64)), cst((1, 64)),
        ],
        out_specs=cst((B, 64)),
        out_shape=jax.ShapeDtypeStruct((B, 64), f32),
        scratch_shapes=[pltpu.VMEM((B, 256), f32)],
        compiler_params=pltpu.CompilerParams(
            dimension_semantics=("arbitrary",)),
    )(hf, Wm1, bm1.reshape(1, 256), Wm2, bm2.reshape(1, 256),
      Wm3, bm3.reshape(1, 256), Wv, bv.reshape(1, 1), Wadv_r, badv.reshape(1, 64))
    return q.reshape(B, 32, 2)


# speedup vs baseline: 3.6316x; 3.6316x over previous
import functools

import jax
import jax.numpy as jnp
from jax import lax
from jax.experimental import pallas as pl
from jax.experimental.pallas import tpu as pltpu

B = 64
N = 128
E = 2048
G = 4


def _graph_body(eit_ref, ei_ref, x_ref,
                W1a_ref, b1a_ref, W1b_ref, b1b_ref,
                W2a_ref, b2a_ref, W2b_ref, b2b_ref,
                W3a_ref, b3a_ref, W3b_ref, b3b_ref,
                g1_ref, be1_ref, g2_ref, be2_ref, g3_ref, be3_ref,
                out_ref, av_s, bv_s):
    f32 = jnp.float32
    src_col = eit_ref[:, 0:1]
    dst_col = eit_ref[:, 1:2]
    dst_row = ei_ref[1:2, :]
    niota_r = lax.broadcasted_iota(jnp.int32, (E, N), 1)
    D = (dst_col == niota_r).astype(f32)
    S = (src_col == niota_r).astype(f32)
    niota_c = lax.broadcasted_iota(jnp.int32, (N, E), 0)
    DT = (niota_c == dst_row).astype(f32)
    deg = jnp.sum(DT, axis=1, keepdims=True)

    def layer(h, fin, Wa_ref, ba_ref, Wb_ref, bb_ref, g_ref, be_ref):
        Wa = Wa_ref[...]
        Wd = Wa[:fin] - Wa[fin:]
        Wbot = Wa[fin:]
        ba = ba_ref[...]
        hf = h.reshape(B * N, fin)
        av_s[...] = (jnp.dot(hf, Wd, preferred_element_type=f32, precision=lax.Precision.HIGHEST) + ba).reshape(B, N, 64)
        bv_s[...] = jnp.dot(hf, Wbot, preferred_element_type=f32, precision=lax.Precision.HIGHEST).reshape(B, N, 64)

        Wb = Wb_ref[...]
        Z = jnp.zeros((64, 128), f32)
        WbG = jnp.concatenate(
            [jnp.concatenate([Wb if j == i else Z for j in range(G)], axis=1)
             for i in range(G)], axis=0)
        bb = bb_ref[...]
        bbG = jnp.concatenate([bb] * G, axis=1)

        def grp(i, carry):
            a = jnp.concatenate(
                [av_s[pl.ds(i * G + j, 1), :, :].reshape(N, 64)
                 for j in range(G)], axis=1)
            bm = jnp.concatenate(
                [bv_s[pl.ds(i * G + j, 1), :, :].reshape(N, 64)
                 for j in range(G)], axis=1)
            pre = (jnp.dot(D, a, preferred_element_type=f32, precision=lax.Precision.HIGHEST)
                   + jnp.dot(S, bm, preferred_element_type=f32, precision=lax.Precision.HIGHEST))
            r = jnp.maximum(pre, 0.0)
            agg = jnp.dot(DT, r, preferred_element_type=f32, precision=lax.Precision.HIGHEST)
            o = jnp.dot(agg, WbG, preferred_element_type=f32, precision=lax.Precision.HIGHEST) + deg * bbG
            for j in range(G):
                out_ref[pl.ds(i * G + j, 1), :, :] = o[:, j * 128:(j + 1) * 128].reshape(1, N, 128)
            return carry

        lax.fori_loop(0, B // G, grp, 0)
        out = out_ref[...]
        inv = 1.0 / (B * 128)
        mu = jnp.sum(jnp.sum(out, axis=0, keepdims=True), axis=2, keepdims=True) * inv
        d = out - mu
        var = jnp.sum(jnp.sum(d * d, axis=0, keepdims=True), axis=2, keepdims=True) * inv
        hn = d * lax.rsqrt(var + 1e-5) * g_ref[...] + be_ref[...]
        return jnp.maximum(hn, 0.0)

    h = layer(x_ref[...], 2, W1a_ref, b1a_ref, W1b_ref, b1b_ref, g1_ref, be1_ref)
    h = layer(h, 128, W2a_ref, b2a_ref, W2b_ref, b2b_ref, g2_ref, be2_ref)
    h = layer(h, 128, W3a_ref, b3a_ref, W3b_ref, b3b_ref, g3_ref, be3_ref)
    out_ref[...] = h


def _head_body(hf_ref, Wm1_ref, bm1_ref, Wm2_ref, bm2_ref, Wm3_ref, bm3_ref,
               Wv_ref, bv_ref, Wadv_ref, badv_ref, out_ref, acc_ref):
    f32 = jnp.float32
    k = pl.program_id(0)

    @pl.when(k == 0)
    def _():
        acc_ref[...] = jnp.zeros_like(acc_ref)

    acc_ref[...] += jnp.dot(hf_ref[...], Wm1_ref[...], preferred_element_type=f32, precision=lax.Precision.HIGHEST)

    @pl.when(k == pl.num_programs(0) - 1)
    def _():
        z = jnp.maximum(acc_ref[...] + bm1_ref[...], 0.0)
        z = jnp.maximum(jnp.dot(z, Wm2_ref[...], preferred_element_type=f32, precision=lax.Precision.HIGHEST) + bm2_ref[...], 0.0)
        z = jnp.maximum(jnp.dot(z, Wm3_ref[...], preferred_element_type=f32, precision=lax.Precision.HIGHEST) + bm3_ref[...], 0.0)
        value = jnp.dot(z, Wv_ref[...], preferred_element_type=f32, precision=lax.Precision.HIGHEST) + bv_ref[...]
        adv = jnp.dot(z, Wadv_ref[...], preferred_element_type=f32, precision=lax.Precision.HIGHEST) + badv_ref[...]
        ii = lax.broadcasted_iota(jnp.int32, (64, 64), 0) // 2
        jj = lax.broadcasted_iota(jnp.int32, (64, 64), 1) // 2
        P = 0.5 * (ii == jj).astype(f32)
        out_ref[...] = value + adv - jnp.dot(adv, P, preferred_element_type=f32, precision=lax.Precision.HIGHEST)


def kernel(x, edge_index, W1a, b1a, W1b, b1b, W2a, b2a, W2b, b2b,
           W3a, b3a, W3b, b3b, g1, be1, g2, be2, g3, be3,
           Wm1, bm1, Wm2, bm2, Wm3, bm3, Wv, bv, Wadv, badv):
    f32 = jnp.float32
    eit = edge_index.T
    h3 = pl.pallas_call(
        _graph_body,
        out_shape=jax.ShapeDtypeStruct((B, N, 128), f32),
        scratch_shapes=[pltpu.VMEM((B, N, 64), f32),
                        pltpu.VMEM((B, N, 64), f32)],
    )(eit, edge_index, x,
      W1a, b1a.reshape(1, 64), W1b, b1b.reshape(1, 128),
      W2a, b2a.reshape(1, 64), W2b, b2b.reshape(1, 128),
      W3a, b3a.reshape(1, 64), W3b, b3b.reshape(1, 128),
      g1.reshape(1, N, 1), be1.reshape(1, N, 1),
      g2.reshape(1, N, 1), be2.reshape(1, N, 1),
      g3.reshape(1, N, 1), be3.reshape(1, N, 1))

    hf = h3.reshape(B, N * N)
    Wadv_r = jnp.transpose(Wadv, (1, 0, 2)).reshape(256, 64)
    KC = 8
    KW = (N * N) // KC

    def cst(shape):
        return pl.BlockSpec(shape, lambda k: tuple(0 for _ in shape))

    q = pl.pallas_call(
        _head_body,
        grid=(KC,),
        in_specs=[
            pl.BlockSpec((B, KW), lambda k: (0, k)),
            pl.BlockSpec((KW, 256), lambda k: (k, 0)),
            cst((1, 256)), cst((256, 256)), cst((1, 256)),
            cst((256, 256)), cst((1, 256)),
            cst((256, 1)), cst((1, 1)), cst((256, 64)), cst((1, 64)),
        ],
        out_specs=cst((B, 64)),
        out_shape=jax.ShapeDtypeStruct((B, 64), f32),
        scratch_shapes=[pltpu.VMEM((B, 256), f32)],
        compiler_params=pltpu.CompilerParams(
            dimension_semantics=("arbitrary",)),
    )(hf, Wm1, bm1.reshape(1, 256), Wm2, bm2.reshape(1, 256),
      Wm3, bm3.reshape(1, 256), Wv, bv.reshape(1, 1), Wadv_r, badv.reshape(1, 64))
    return q.reshape(B, 32, 2)


# speedup vs baseline: 9.0140x; 2.4821x over previous
import functools

import jax
import jax.numpy as jnp
from jax import lax
from jax.experimental import pallas as pl
from jax.experimental.pallas import tpu as pltpu

B = 64
N = 128
E = 2048
G = 4


def _split(x):
    hi = x.astype(jnp.bfloat16)
    lo = (x - hi.astype(jnp.float32)).astype(jnp.bfloat16)
    return hi, lo


def _dot_oh(oh_bf16, x):
    hi, lo = _split(x)
    return (jnp.dot(oh_bf16, hi, preferred_element_type=jnp.float32)
            + jnp.dot(oh_bf16, lo, preferred_element_type=jnp.float32))


def _dot3(x, w):
    xh, xl = _split(x)
    wh, wl = _split(w)
    return (jnp.dot(xh, wh, preferred_element_type=jnp.float32)
            + jnp.dot(xh, wl, preferred_element_type=jnp.float32)
            + jnp.dot(xl, wh, preferred_element_type=jnp.float32))


def _dot3_pre(x, wh, wl):
    xh, xl = _split(x)
    return (jnp.dot(xh, wh, preferred_element_type=jnp.float32)
            + jnp.dot(xh, wl, preferred_element_type=jnp.float32)
            + jnp.dot(xl, wh, preferred_element_type=jnp.float32))


def _graph_body(eit_ref, ei_ref, x_ref,
                W1a_ref, b1a_ref, W1b_ref, b1b_ref,
                W2a_ref, b2a_ref, W2b_ref, b2b_ref,
                W3a_ref, b3a_ref, W3b_ref, b3b_ref,
                g1_ref, be1_ref, g2_ref, be2_ref, g3_ref, be3_ref,
                out_ref, av_s, bv_s):
    f32 = jnp.float32
    src_col = eit_ref[:, 0:1]
    dst_col = eit_ref[:, 1:2]
    dst_row = ei_ref[1:2, :]
    bf16 = jnp.bfloat16
    niota_r = lax.broadcasted_iota(jnp.int32, (E, N), 1)
    DS = jnp.concatenate([(dst_col == niota_r).astype(bf16),
                          (src_col == niota_r).astype(bf16)], axis=1)
    niota_c = lax.broadcasted_iota(jnp.int32, (N, E), 0)
    dmask = (niota_c == dst_row)
    DT = dmask.astype(bf16)
    deg = jnp.sum(dmask.astype(f32), axis=1, keepdims=True)

    def layer(h, fin, Wa_ref, ba_ref, Wb_ref, bb_ref, g_ref, be_ref):
        Wa = Wa_ref[...]
        Wd = Wa[:fin] - Wa[fin:]
        Wbot = Wa[fin:]
        ba = ba_ref[...]
        hf = h.reshape(B * N, fin)
        if fin == 2:
            av_s[...] = (jnp.dot(hf, Wd, preferred_element_type=f32, precision=lax.Precision.HIGHEST) + ba).reshape(B, N, 64)
            bv_s[...] = jnp.dot(hf, Wbot, preferred_element_type=f32, precision=lax.Precision.HIGHEST).reshape(B, N, 64)
        else:
            av_s[...] = (_dot3(hf, Wd) + ba).reshape(B, N, 64)
            bv_s[...] = _dot3(hf, Wbot).reshape(B, N, 64)

        Wb = Wb_ref[...]
        Z = jnp.zeros((64, 128), f32)
        WbG = jnp.concatenate(
            [jnp.concatenate([Wb if j == i else Z for j in range(G)], axis=1)
             for i in range(G)], axis=0)
        WbGh, WbGl = _split(WbG)
        bb = bb_ref[...]
        bbG = jnp.concatenate([bb] * G, axis=1)

        def grp(i, carry):
            a = jnp.concatenate(
                [av_s[pl.ds(i * G + j, 1), :, :].reshape(N, 64)
                 for j in range(G)], axis=1)
            bm = jnp.concatenate(
                [bv_s[pl.ds(i * G + j, 1), :, :].reshape(N, 64)
                 for j in range(G)], axis=1)
            pre = _dot_oh(DS, jnp.concatenate([a, bm], axis=0))
            r = jnp.maximum(pre, 0.0)
            agg = _dot_oh(DT, r)
            o = _dot3_pre(agg, WbGh, WbGl) + deg * bbG
            for j in range(G):
                out_ref[pl.ds(i * G + j, 1), :, :] = o[:, j * 128:(j + 1) * 128].reshape(1, N, 128)
            return carry

        lax.fori_loop(0, B // G, grp, 0)
        out = out_ref[...]
        inv = 1.0 / (B * 128)
        mu = jnp.sum(jnp.sum(out, axis=0, keepdims=True), axis=2, keepdims=True) * inv
        d = out - mu
        var = jnp.sum(jnp.sum(d * d, axis=0, keepdims=True), axis=2, keepdims=True) * inv
        hn = d * lax.rsqrt(var + 1e-5) * g_ref[...] + be_ref[...]
        return jnp.maximum(hn, 0.0)

    h = layer(x_ref[...], 2, W1a_ref, b1a_ref, W1b_ref, b1b_ref, g1_ref, be1_ref)
    h = layer(h, 128, W2a_ref, b2a_ref, W2b_ref, b2b_ref, g2_ref, be2_ref)
    h = layer(h, 128, W3a_ref, b3a_ref, W3b_ref, b3b_ref, g3_ref, be3_ref)
    out_ref[...] = h


def _head_body(hf_ref, Wm1_ref, bm1_ref, Wm2_ref, bm2_ref, Wm3_ref, bm3_ref,
               Wv_ref, bv_ref, Wadv_ref, badv_ref, out_ref, acc_ref):
    f32 = jnp.float32
    k = pl.program_id(0)

    @pl.when(k == 0)
    def _():
        acc_ref[...] = jnp.zeros_like(acc_ref)

    acc_ref[...] += _dot3(hf_ref[...], Wm1_ref[...])

    @pl.when(k == pl.num_programs(0) - 1)
    def _():
        z = jnp.maximum(acc_ref[...] + bm1_ref[...], 0.0)
        z = jnp.maximum(jnp.dot(z, Wm2_ref[...], preferred_element_type=f32, precision=lax.Precision.HIGHEST) + bm2_ref[...], 0.0)
        z = jnp.maximum(jnp.dot(z, Wm3_ref[...], preferred_element_type=f32, precision=lax.Precision.HIGHEST) + bm3_ref[...], 0.0)
        value = jnp.dot(z, Wv_ref[...], preferred_element_type=f32, precision=lax.Precision.HIGHEST) + bv_ref[...]
        adv = jnp.dot(z, Wadv_ref[...], preferred_element_type=f32, precision=lax.Precision.HIGHEST) + badv_ref[...]
        ii = lax.broadcasted_iota(jnp.int32, (64, 64), 0) // 2
        jj = lax.broadcasted_iota(jnp.int32, (64, 64), 1) // 2
        P = 0.5 * (ii == jj).astype(f32)
        out_ref[...] = value + adv - jnp.dot(adv, P, preferred_element_type=f32, precision=lax.Precision.HIGHEST)


def kernel(x, edge_index, W1a, b1a, W1b, b1b, W2a, b2a, W2b, b2b,
           W3a, b3a, W3b, b3b, g1, be1, g2, be2, g3, be3,
           Wm1, bm1, Wm2, bm2, Wm3, bm3, Wv, bv, Wadv, badv):
    f32 = jnp.float32
    eit = edge_index.T
    h3 = pl.pallas_call(
        _graph_body,
        out_shape=jax.ShapeDtypeStruct((B, N, 128), f32),
        scratch_shapes=[pltpu.VMEM((B, N, 64), f32),
                        pltpu.VMEM((B, N, 64), f32)],
    )(eit, edge_index, x,
      W1a, b1a.reshape(1, 64), W1b, b1b.reshape(1, 128),
      W2a, b2a.reshape(1, 64), W2b, b2b.reshape(1, 128),
      W3a, b3a.reshape(1, 64), W3b, b3b.reshape(1, 128),
      g1.reshape(1, N, 1), be1.reshape(1, N, 1),
      g2.reshape(1, N, 1), be2.reshape(1, N, 1),
      g3.reshape(1, N, 1), be3.reshape(1, N, 1))

    hf = h3.reshape(B, N * N)
    Wadv_r = jnp.transpose(Wadv, (1, 0, 2)).reshape(256, 64)
    KC = 8
    KW = (N * N) // KC

    def cst(shape):
        return pl.BlockSpec(shape, lambda k: tuple(0 for _ in shape))

    q = pl.pallas_call(
        _head_body,
        grid=(KC,),
        in_specs=[
            pl.BlockSpec((B, KW), lambda k: (0, k)),
            pl.BlockSpec((KW, 256), lambda k: (k, 0)),
            cst((1, 256)), cst((256, 256)), cst((1, 256)),
            cst((256, 256)), cst((1, 256)),
            cst((256, 1)), cst((1, 1)), cst((256, 64)), cst((1, 64)),
        ],
        out_specs=cst((B, 64)),
        out_shape=jax.ShapeDtypeStruct((B, 64), f32),
        scratch_shapes=[pltpu.VMEM((B, 256), f32)],
        compiler_params=pltpu.CompilerParams(
            dimension_semantics=("arbitrary",)),
    )(hf, Wm1, bm1.reshape(1, 256), Wm2, bm2.reshape(1, 256),
      Wm3, bm3.reshape(1, 256), Wv, bv.reshape(1, 1), Wadv_r, badv.reshape(1, 64))
    return q.reshape(B, 32, 2)
